# baseline (speedup 1.0000x reference)
import functools

import jax
import jax.numpy as jnp
from jax import lax
from jax.experimental import pallas as pl
from jax.experimental.pallas import tpu as pltpu
from jax.experimental.pallas import tpu_sc as plsc

B = 16384
D = 128
K = 20
NC = 2
NS = 16
NW = NC * NS
SPW = B // NW
C = 16
NCHUNK = SPW // C
NPER = 64
NSUB = C * K // NPER
L = 16
G = D // L
SPL = 128 // L


def _make_sc_partials():
    mesh = plsc.VectorSubcoreMesh(core_axis_name="c", subcore_axis_name="s")

    @functools.partial(
        pl.kernel,
        out_type=(
            jax.ShapeDtypeStruct((B * L,), jnp.float32),
            jax.ShapeDtypeStruct((B * L,), jnp.float32),
        ),
        mesh=mesh,
        scratch_types=[
            pltpu.VMEM((SPW,), jnp.int32),
            pltpu.VMEM((SPW,), jnp.int32),
            pltpu.VMEM((SPW * K,), jnp.int32),
            pltpu.VMEM((2, C, D), jnp.float32),
            pltpu.VMEM((2, C, D), jnp.float32),
            pltpu.VMEM((2, C * K, D), jnp.float32),
            pltpu.VMEM((SPW * L,), jnp.float32),
            pltpu.VMEM((SPW * L,), jnp.float32),
            pltpu.SemaphoreType.DMA,
            pltpu.SemaphoreType.DMA,
        ],
    )
    def sc_partials(doc_u, pos_v, neg_flat, u_table, v_table, out1, out2,
                    idxu_all, idxp_all, idxn_all, u_rows, p_rows, n_rows,
                    out1_flat, out2_flat, sem0, sem1):
        wid = lax.axis_index("c") * NS + lax.axis_index("s")
        wbase = wid * SPW
        sems = (sem0, sem1)

        pltpu.sync_copy(doc_u.at[pl.ds(wbase, SPW)], idxu_all)
        pltpu.sync_copy(pos_v.at[pl.ds(wbase, SPW)], idxp_all)
        pltpu.sync_copy(neg_flat.at[pl.ds(wbase * K, SPW * K)], idxn_all)

        def issue(c, slot):
            sem = sems[slot]
            pltpu.async_copy(
                u_table.at[idxu_all.at[pl.ds(c * C, C)]], u_rows.at[slot], sem)
            pltpu.async_copy(
                v_table.at[idxp_all.at[pl.ds(c * C, C)]], p_rows.at[slot], sem)
            for j in range(NSUB):
                pltpu.async_copy(
                    v_table.at[idxn_all.at[pl.ds(c * C * K + j * NPER, NPER)]],
                    n_rows.at[slot].at[pl.ds(j * NPER, NPER)], sem)

        def drain(slot):
            sem = sems[slot]
            pltpu.make_async_copy(
                u_table.at[pl.ds(0, C)], u_rows.at[slot], sem).wait()
            pltpu.make_async_copy(
                v_table.at[pl.ds(0, C)], p_rows.at[slot], sem).wait()
            pltpu.make_async_copy(
                v_table.at[pl.ds(0, C * K)], n_rows.at[slot], sem).wait()

        def compute(c, slot):
            ur = u_rows.at[slot]
            pr = p_rows.at[slot]
            nr = n_rows.at[slot]

            @plsc.parallel_loop(0, C, 1, unroll=1)
            def _(i):
                u = [ur[i, pl.ds(g * L, L)] for g in range(G)]
                acc1 = u[0] * pr[i, pl.ds(0, L)]
                for g in range(1, G):
                    acc1 = acc1 + u[g] * pr[i, pl.ds(g * L, L)]
                nacc = [nr[i * K, pl.ds(g * L, L)] for g in range(G)]
                for k in range(1, K):
                    for g in range(G):
                        nacc[g] = nacc[g] + nr[i * K + k, pl.ds(g * L, L)]
                acc2 = u[0] * nacc[0]
                for g in range(1, G):
                    acc2 = acc2 + u[g] * nacc[g]
                out1_flat[pl.ds((c * C + i) * L, L)] = acc1
                out2_flat[pl.ds((c * C + i) * L, L)] = acc2

        issue(0, 0)

        def pair_body(i, carry):
            issue(2 * i + 1, 1)
            drain(0)
            compute(2 * i, 0)

            @pl.when(i < NCHUNK // 2 - 1)
            def _():
                issue(2 * i + 2, 0)

            drain(1)
            compute(2 * i + 1, 1)
            return carry

        lax.fori_loop(0, NCHUNK // 2, pair_body, 0)
        pltpu.sync_copy(out1_flat, out1.at[pl.ds(wbase * L, SPW * L)])
        pltpu.sync_copy(out2_flat, out2.at[pl.ds(wbase * L, SPW * L)])

    return sc_partials


_sc_partials = _make_sc_partials()


def _tc_loss_body(p1_ref, p2_ref, out_ref):
    d = lax.broadcasted_iota(jnp.int32, (128, 128), 0)
    j = lax.broadcasted_iota(jnp.int32, (128, 128), 1)
    w = jnp.where(d // L == j, 1.0, 0.0).astype(jnp.float32)
    s1 = jax.lax.dot(p1_ref[...], w, precision=jax.lax.Precision.HIGHEST)
    s2 = jax.lax.dot(p2_ref[...], w, precision=jax.lax.Precision.HIGHEST)
    valid = lax.broadcasted_iota(jnp.int32, s1.shape, 1) < SPL
    contrib = jnp.where(
        valid, jax.nn.log_sigmoid(s1) + jax.nn.log_sigmoid(-s2), 0.0)
    out_ref[...] = jnp.broadcast_to(-jnp.sum(contrib), (1, 1))


def kernel(doc_u, pos_v, neg_v, u_table, v_table, d_table):
    del d_table
    neg_flat = neg_v.reshape(B * K)
    p1, p2 = _sc_partials(doc_u, pos_v, neg_flat, u_table, v_table)
    loss = pl.pallas_call(
        _tc_loss_body,
        out_shape=jax.ShapeDtypeStruct((1, 1), jnp.float32),
    )(p1.reshape(B * L // 128, 128), p2.reshape(B * L // 128, 128))
    return loss[0, 0]


# speedup vs baseline: 1.3042x; 1.3042x over previous
import functools

import jax
import jax.numpy as jnp
from jax import lax
from jax.experimental import pallas as pl
from jax.experimental.pallas import tpu as pltpu
from jax.experimental.pallas import tpu_sc as plsc

B = 16384
D = 128
K = 20
NC = 2
NS = 16
NW = NC * NS
SPW = B // NW
C = 64
NCHUNK = SPW // C
L = 16
G = D // L
SPL = 128 // L


def _make_sc_partials():
    mesh = plsc.VectorSubcoreMesh(core_axis_name="c", subcore_axis_name="s")

    @functools.partial(
        pl.kernel,
        out_type=(
            jax.ShapeDtypeStruct((B * L,), jnp.float32),
            jax.ShapeDtypeStruct((B * L,), jnp.float32),
        ),
        mesh=mesh,
        scratch_types=[
            pltpu.VMEM((SPW,), jnp.int32),
            pltpu.VMEM((SPW,), jnp.int32),
            pltpu.VMEM((SPW * K,), jnp.int32),
            pltpu.VMEM((2, C, D), jnp.float32),
            pltpu.VMEM((2, C, D), jnp.float32),
            pltpu.VMEM((2, C, D), jnp.float32),
            pltpu.VMEM((SPW * L,), jnp.float32),
            pltpu.VMEM((SPW * L,), jnp.float32),
            pltpu.SemaphoreType.DMA,
            pltpu.SemaphoreType.DMA,
        ],
    )
    def sc_partials(doc_u, pos_v, neg_kmaj, u_table, v_table, out1, out2,
                    idxu_all, idxp_all, idxn_all, u_rows, p_rows, nsum,
                    out1_flat, out2_flat, sem0, sem1):
        wid = lax.axis_index("c") * NS + lax.axis_index("s")
        wbase = wid * SPW
        sems = (sem0, sem1)

        pltpu.sync_copy(doc_u.at[pl.ds(wbase, SPW)], idxu_all)
        pltpu.sync_copy(pos_v.at[pl.ds(wbase, SPW)], idxp_all)
        for k in range(K):
            pltpu.sync_copy(neg_kmaj.at[pl.ds(k * B + wbase, SPW)],
                            idxn_all.at[pl.ds(k * SPW, SPW)])

        zero = jnp.zeros((L,), jnp.float32)

        def zero_rows(slot):
            def zrow(i, carry):
                for g in range(G):
                    nsum[slot, i, pl.ds(g * L, L)] = zero
                return carry
            lax.fori_loop(0, C, zrow, 0)

        def issue(c, slot):
            sem = sems[slot]
            pltpu.async_copy(
                u_table.at[idxu_all.at[pl.ds(c * C, C)]], u_rows.at[slot], sem)
            pltpu.async_copy(
                v_table.at[idxp_all.at[pl.ds(c * C, C)]], p_rows.at[slot], sem)
            for k in range(K):
                pltpu.async_copy(
                    v_table.at[idxn_all.at[pl.ds(k * SPW + c * C, C)]],
                    nsum.at[slot], sem, add=True)

        def drain(slot):
            sem = sems[slot]
            pltpu.make_async_copy(
                u_table.at[pl.ds(0, C)], u_rows.at[slot], sem).wait()
            pltpu.make_async_copy(
                v_table.at[pl.ds(0, C)], p_rows.at[slot], sem).wait()
            for _ in range(K):
                pltpu.make_async_copy(
                    v_table.at[pl.ds(0, C)], nsum.at[slot], sem).wait()

        def compute(c, slot):
            ur = u_rows.at[slot]
            pr = p_rows.at[slot]
            nr = nsum.at[slot]

            def sample_body(i, carry):
                u = [ur[i, pl.ds(g * L, L)] for g in range(G)]
                acc1 = u[0] * pr[i, pl.ds(0, L)]
                for g in range(1, G):
                    acc1 = acc1 + u[g] * pr[i, pl.ds(g * L, L)]
                acc2 = u[0] * nr[i, pl.ds(0, L)]
                for g in range(1, G):
                    acc2 = acc2 + u[g] * nr[i, pl.ds(g * L, L)]
                for g in range(G):
                    nr[i, pl.ds(g * L, L)] = zero
                out1_flat[pl.ds((c * C + i) * L, L)] = acc1
                out2_flat[pl.ds((c * C + i) * L, L)] = acc2
                return carry

            lax.fori_loop(0, C, sample_body, 0)

        zero_rows(0)
        zero_rows(1)
        issue(0, 0)

        def pair_body(i, carry):
            issue(2 * i + 1, 1)
            drain(0)
            compute(2 * i, 0)

            @pl.when(i < NCHUNK // 2 - 1)
            def _():
                issue(2 * i + 2, 0)

            drain(1)
            compute(2 * i + 1, 1)
            return carry

        lax.fori_loop(0, NCHUNK // 2, pair_body, 0)
        pltpu.sync_copy(out1_flat, out1.at[pl.ds(wbase * L, SPW * L)])
        pltpu.sync_copy(out2_flat, out2.at[pl.ds(wbase * L, SPW * L)])

    return sc_partials


_sc_partials = _make_sc_partials()


def _tc_loss_body(p1_ref, p2_ref, out_ref):
    d = lax.broadcasted_iota(jnp.int32, (128, 128), 0)
    j = lax.broadcasted_iota(jnp.int32, (128, 128), 1)
    w = jnp.where(d // L == j, 1.0, 0.0).astype(jnp.float32)
    s1 = jax.lax.dot(p1_ref[...], w, precision=jax.lax.Precision.HIGHEST)
    s2 = jax.lax.dot(p2_ref[...], w, precision=jax.lax.Precision.HIGHEST)
    valid = lax.broadcasted_iota(jnp.int32, s1.shape, 1) < SPL
    contrib = jnp.where(
        valid, jax.nn.log_sigmoid(s1) + jax.nn.log_sigmoid(-s2), 0.0)
    out_ref[...] = jnp.broadcast_to(-jnp.sum(contrib), (1, 1))


def kernel(doc_u, pos_v, neg_v, u_table, v_table, d_table):
    del d_table
    neg_kmaj = neg_v.T.reshape(B * K)
    p1, p2 = _sc_partials(doc_u, pos_v, neg_kmaj, u_table, v_table)
    loss = pl.pallas_call(
        _tc_loss_body,
        out_shape=jax.ShapeDtypeStruct((1, 1), jnp.float32),
    )(p1.reshape(B * L // 128, 128), p2.reshape(B * L // 128, 128))
    return loss[0, 0]


# speedup vs baseline: 1.4468x; 1.1094x over previous
import functools

import jax
import jax.numpy as jnp
from jax import lax
from jax.experimental import pallas as pl
from jax.experimental.pallas import tpu as pltpu
from jax.experimental.pallas import tpu_sc as plsc

B = 16384
D = 128
K = 20
NC = 2
NS = 16
NW = NC * NS
SPW = B // NW
C = 64
NCHUNK = SPW // C
L = 16
G = D // L
SPL = 128 // L


def _make_sc_partials():
    mesh = plsc.VectorSubcoreMesh(core_axis_name="c", subcore_axis_name="s")

    @functools.partial(
        pl.kernel,
        out_type=(
            jax.ShapeDtypeStruct((B * L,), jnp.float32),
            jax.ShapeDtypeStruct((B * L,), jnp.float32),
        ),
        mesh=mesh,
        scratch_types=[
            pltpu.VMEM((SPW,), jnp.int32),
            pltpu.VMEM((SPW,), jnp.int32),
            pltpu.VMEM((SPW * K,), jnp.int32),
            pltpu.VMEM((2, C, D), jnp.float32),
            pltpu.VMEM((2, C, D), jnp.float32),
            pltpu.VMEM((2, C, D), jnp.float32),
            pltpu.VMEM((SPW * L,), jnp.float32),
            pltpu.VMEM((SPW * L,), jnp.float32),
            pltpu.SemaphoreType.DMA,
            pltpu.SemaphoreType.DMA,
        ],
    )
    def sc_partials(doc_u, pos_v, neg_kmaj, u_table, v_table, out1, out2,
                    idxu_all, idxp_all, idxn_all, u_rows, p_rows, nsum,
                    out1_flat, out2_flat, sem0, sem1):
        wid = lax.axis_index("c") * NS + lax.axis_index("s")
        wbase = wid * SPW
        sems = (sem0, sem1)

        pltpu.async_copy(doc_u.at[pl.ds(wbase, SPW)], idxu_all, sem0)
        pltpu.async_copy(pos_v.at[pl.ds(wbase, SPW)], idxp_all, sem0)
        for k in range(K):
            pltpu.async_copy(neg_kmaj.at[pl.ds(k * B + wbase, SPW)],
                             idxn_all.at[pl.ds(k * SPW, SPW)], sem0)
        pltpu.make_async_copy(doc_u.at[pl.ds(wbase, SPW)], idxu_all,
                              sem0).wait()
        pltpu.make_async_copy(pos_v.at[pl.ds(wbase, SPW)], idxp_all,
                              sem0).wait()
        pltpu.make_async_copy(neg_kmaj.at[pl.ds(0, SPW * K)], idxn_all,
                              sem0).wait()

        zero = jnp.zeros((L,), jnp.float32)

        def zero_rows(slot):
            def zrow(i, carry):
                for g in range(G):
                    nsum[slot, i, pl.ds(g * L, L)] = zero
                return carry
            lax.fori_loop(0, C, zrow, 0)

        def issue(c, slot):
            sem = sems[slot]
            pltpu.async_copy(
                u_table.at[idxu_all.at[pl.ds(c * C, C)]], u_rows.at[slot], sem)
            pltpu.async_copy(
                v_table.at[idxp_all.at[pl.ds(c * C, C)]], p_rows.at[slot], sem)
            for k in range(K):
                pltpu.async_copy(
                    v_table.at[idxn_all.at[pl.ds(k * SPW + c * C, C)]],
                    nsum.at[slot], sem, add=True)

        def drain(slot):
            sem = sems[slot]
            pltpu.make_async_copy(
                u_table.at[pl.ds(0, C)], u_rows.at[slot], sem).wait()
            pltpu.make_async_copy(
                v_table.at[pl.ds(0, C)], p_rows.at[slot], sem).wait()
            for _ in range(K):
                pltpu.make_async_copy(
                    v_table.at[pl.ds(0, C)], nsum.at[slot], sem).wait()

        def compute(c, slot):
            ur = u_rows.at[slot]
            pr = p_rows.at[slot]
            nr = nsum.at[slot]

            def sample_body(i, carry):
                u = [ur[i, pl.ds(g * L, L)] for g in range(G)]
                acc1 = u[0] * pr[i, pl.ds(0, L)]
                for g in range(1, G):
                    acc1 = acc1 + u[g] * pr[i, pl.ds(g * L, L)]
                acc2 = u[0] * nr[i, pl.ds(0, L)]
                for g in range(1, G):
                    acc2 = acc2 + u[g] * nr[i, pl.ds(g * L, L)]
                for g in range(G):
                    nr[i, pl.ds(g * L, L)] = zero
                out1_flat[pl.ds((c * C + i) * L, L)] = acc1
                out2_flat[pl.ds((c * C + i) * L, L)] = acc2
                return carry

            lax.fori_loop(0, C, sample_body, 0)

        zero_rows(0)
        zero_rows(1)
        issue(0, 0)

        def pair_body(i, carry):
            issue(2 * i + 1, 1)
            drain(0)
            compute(2 * i, 0)

            @pl.when(i < NCHUNK // 2 - 1)
            def _():
                issue(2 * i + 2, 0)

            drain(1)
            compute(2 * i + 1, 1)
            return carry

        lax.fori_loop(0, NCHUNK // 2, pair_body, 0)
        pltpu.async_copy(out1_flat, out1.at[pl.ds(wbase * L, SPW * L)], sem0)
        pltpu.async_copy(out2_flat, out2.at[pl.ds(wbase * L, SPW * L)], sem0)
        pltpu.make_async_copy(
            out1_flat, out1.at[pl.ds(wbase * L, SPW * L)], sem0).wait()
        pltpu.make_async_copy(
            out2_flat, out2.at[pl.ds(wbase * L, SPW * L)], sem0).wait()

    return sc_partials


_sc_partials = _make_sc_partials()


def _tc_loss_body(p1_ref, p2_ref, out_ref):
    d = lax.broadcasted_iota(jnp.int32, (128, 128), 0)
    j = lax.broadcasted_iota(jnp.int32, (128, 128), 1)
    w = jnp.where(d // L == j, 1.0, 0.0).astype(jnp.float32)
    s1 = jax.lax.dot(p1_ref[...], w, precision=jax.lax.Precision.HIGHEST)
    s2 = jax.lax.dot(p2_ref[...], w, precision=jax.lax.Precision.HIGHEST)
    valid = lax.broadcasted_iota(jnp.int32, s1.shape, 1) < SPL
    contrib = jnp.where(
        valid, jax.nn.log_sigmoid(s1) + jax.nn.log_sigmoid(-s2), 0.0)
    out_ref[...] = jnp.broadcast_to(-jnp.sum(contrib), (1, 1))


def kernel(doc_u, pos_v, neg_v, u_table, v_table, d_table):
    del d_table
    neg_kmaj = neg_v.T.reshape(B * K)
    p1, p2 = _sc_partials(doc_u, pos_v, neg_kmaj, u_table, v_table)
    loss = pl.pallas_call(
        _tc_loss_body,
        out_shape=jax.ShapeDtypeStruct((1, 1), jnp.float32),
    )(p1.reshape(B * L // 128, 128), p2.reshape(B * L // 128, 128))
    return loss[0, 0]


# speedup vs baseline: 1.4653x; 1.0128x over previous
import functools

import jax
import jax.numpy as jnp
from jax import lax
from jax.experimental import pallas as pl
from jax.experimental.pallas import tpu as pltpu
from jax.experimental.pallas import tpu_sc as plsc

B = 16384
D = 128
K = 20
NC = 2
NS = 16
NW = NC * NS
SPW = B // NW
C = 128
NCHUNK = SPW // C
L = 16
G = D // L
SPL = 128 // L


def _make_sc_partials():
    mesh = plsc.VectorSubcoreMesh(core_axis_name="c", subcore_axis_name="s")

    @functools.partial(
        pl.kernel,
        out_type=(
            jax.ShapeDtypeStruct((B * L,), jnp.float32),
            jax.ShapeDtypeStruct((B * L,), jnp.float32),
        ),
        mesh=mesh,
        scratch_types=[
            pltpu.VMEM((SPW,), jnp.int32),
            pltpu.VMEM((SPW,), jnp.int32),
            pltpu.VMEM((SPW * K,), jnp.int32),
            pltpu.VMEM((2, C, D), jnp.float32),
            pltpu.VMEM((2, C, D), jnp.float32),
            pltpu.VMEM((2, C, D), jnp.float32),
            pltpu.VMEM((SPW * L,), jnp.float32),
            pltpu.VMEM((SPW * L,), jnp.float32),
            pltpu.SemaphoreType.DMA,
            pltpu.SemaphoreType.DMA,
        ],
    )
    def sc_partials(doc_u, pos_v, neg_kmaj, u_table, v_table, out1, out2,
                    idxu_all, idxp_all, idxn_all, u_rows, p_rows, nsum,
                    out1_flat, out2_flat, sem0, sem1):
        wid = lax.axis_index("c") * NS + lax.axis_index("s")
        wbase = wid * SPW
        sems = (sem0, sem1)

        pltpu.async_copy(doc_u.at[pl.ds(wbase, SPW)], idxu_all, sem0)
        pltpu.async_copy(pos_v.at[pl.ds(wbase, SPW)], idxp_all, sem0)
        for k in range(K):
            pltpu.async_copy(neg_kmaj.at[pl.ds(k * B + wbase, SPW)],
                             idxn_all.at[pl.ds(k * SPW, SPW)], sem0)
        pltpu.make_async_copy(doc_u.at[pl.ds(wbase, SPW)], idxu_all,
                              sem0).wait()
        pltpu.make_async_copy(pos_v.at[pl.ds(wbase, SPW)], idxp_all,
                              sem0).wait()
        pltpu.make_async_copy(neg_kmaj.at[pl.ds(0, SPW * K)], idxn_all,
                              sem0).wait()

        zero = jnp.zeros((L,), jnp.float32)

        def zero_rows(slot):
            def zrow(i, carry):
                for g in range(G):
                    nsum[slot, i, pl.ds(g * L, L)] = zero
                return carry
            lax.fori_loop(0, C, zrow, 0)

        def issue(c, slot):
            sem = sems[slot]
            pltpu.async_copy(
                u_table.at[idxu_all.at[pl.ds(c * C, C)]], u_rows.at[slot], sem)
            pltpu.async_copy(
                v_table.at[idxp_all.at[pl.ds(c * C, C)]], p_rows.at[slot], sem)
            for k in range(K):
                pltpu.async_copy(
                    v_table.at[idxn_all.at[pl.ds(k * SPW + c * C, C)]],
                    nsum.at[slot], sem, add=True)

        def drain(slot):
            sem = sems[slot]
            pltpu.make_async_copy(
                u_table.at[pl.ds(0, C)], u_rows.at[slot], sem).wait()
            pltpu.make_async_copy(
                v_table.at[pl.ds(0, C)], p_rows.at[slot], sem).wait()
            for _ in range(K):
                pltpu.make_async_copy(
                    v_table.at[pl.ds(0, C)], nsum.at[slot], sem).wait()

        def compute(c, slot):
            ur = u_rows.at[slot]
            pr = p_rows.at[slot]
            nr = nsum.at[slot]

            def sample_body(i, carry):
                u = [ur[i, pl.ds(g * L, L)] for g in range(G)]
                acc1 = u[0] * pr[i, pl.ds(0, L)]
                for g in range(1, G):
                    acc1 = acc1 + u[g] * pr[i, pl.ds(g * L, L)]
                acc2 = u[0] * nr[i, pl.ds(0, L)]
                for g in range(1, G):
                    acc2 = acc2 + u[g] * nr[i, pl.ds(g * L, L)]
                for g in range(G):
                    nr[i, pl.ds(g * L, L)] = zero
                out1_flat[pl.ds((c * C + i) * L, L)] = acc1
                out2_flat[pl.ds((c * C + i) * L, L)] = acc2
                return carry

            lax.fori_loop(0, C, sample_body, 0)

        zero_rows(0)
        zero_rows(1)
        issue(0, 0)

        def pair_body(i, carry):
            issue(2 * i + 1, 1)
            drain(0)
            compute(2 * i, 0)

            @pl.when(i < NCHUNK // 2 - 1)
            def _():
                issue(2 * i + 2, 0)

            drain(1)
            compute(2 * i + 1, 1)
            return carry

        lax.fori_loop(0, NCHUNK // 2, pair_body, 0)
        pltpu.async_copy(out1_flat, out1.at[pl.ds(wbase * L, SPW * L)], sem0)
        pltpu.async_copy(out2_flat, out2.at[pl.ds(wbase * L, SPW * L)], sem0)
        pltpu.make_async_copy(
            out1_flat, out1.at[pl.ds(wbase * L, SPW * L)], sem0).wait()
        pltpu.make_async_copy(
            out2_flat, out2.at[pl.ds(wbase * L, SPW * L)], sem0).wait()

    return sc_partials


_sc_partials = _make_sc_partials()


def _tc_loss_body(p1_ref, p2_ref, out_ref):
    d = lax.broadcasted_iota(jnp.int32, (128, 128), 0)
    j = lax.broadcasted_iota(jnp.int32, (128, 128), 1)
    w = jnp.where(d // L == j, 1.0, 0.0).astype(jnp.float32)
    s1 = jax.lax.dot(p1_ref[...], w, precision=jax.lax.Precision.HIGHEST)
    s2 = jax.lax.dot(p2_ref[...], w, precision=jax.lax.Precision.HIGHEST)
    valid = lax.broadcasted_iota(jnp.int32, s1.shape, 1) < SPL
    contrib = jnp.where(
        valid, jax.nn.log_sigmoid(s1) + jax.nn.log_sigmoid(-s2), 0.0)
    out_ref[...] = jnp.broadcast_to(-jnp.sum(contrib), (1, 1))


def kernel(doc_u, pos_v, neg_v, u_table, v_table, d_table):
    del d_table
    neg_kmaj = neg_v.T.reshape(B * K)
    p1, p2 = _sc_partials(doc_u, pos_v, neg_kmaj, u_table, v_table)
    loss = pl.pallas_call(
        _tc_loss_body,
        out_shape=jax.ShapeDtypeStruct((1, 1), jnp.float32),
    )(p1.reshape(B * L // 128, 128), p2.reshape(B * L // 128, 128))
    return loss[0, 0]
